# baseline (speedup 1.0000x reference)
import functools

import jax
import jax.numpy as jnp
from jax import lax
from jax.experimental import pallas as pl
from jax.experimental.pallas import tpu as pltpu
from jax.experimental.pallas import tpu_sc as plsc

EPS = 1e-5
NC = 2
NS = 16
L = 16


def _rsqrt16(x):
    i = plsc.bitcast(x, jnp.int32)
    y = plsc.bitcast(jnp.int32(0x5F3759DF) - (i >> 1), jnp.float32)
    for _ in range(3):
        y = y * (1.5 - 0.5 * x * y * y)
    return jnp.where(x > 0, y, 0.0)


def _zero_ref(ref, nwords):
    zero16 = jnp.zeros((L,), jnp.float32)

    def zb(i, _):
        ref[pl.ds(i * L, L)] = zero16
        return 0

    lax.fori_loop(0, nwords // L, zb, 0)


def _make_deg(e, npad, ept):
    mesh = plsc.VectorSubcoreMesh(core_axis_name="c", subcore_axis_name="s", num_cores=NC, num_subcores=NS)

    @functools.partial(
        pl.kernel,
        mesh=mesh,
        compiler_params=pltpu.CompilerParams(needs_layout_passes=False),
        out_type=[
            jax.ShapeDtypeStruct((NC * NS * npad,), jnp.float32),
            jax.ShapeDtypeStruct((NC * NS * npad,), jnp.float32),
        ],
        scratch_types=[
            pltpu.VMEM((ept,), jnp.int32),
            pltpu.VMEM((ept,), jnp.int32),
            pltpu.VMEM((ept,), jnp.float32),
            pltpu.VMEM((npad,), jnp.float32),
            pltpu.VMEM((npad,), jnp.float32),
        ],
    )
    def deg(src_h, dst_h, ew_h, dego_h, degi_h, srcb, dstb, ewb, dego, degi):
        c = lax.axis_index("c")
        s = lax.axis_index("s")
        w = s * NC + c
        base = w * ept
        pltpu.sync_copy(src_h.at[pl.ds(base, ept)], srcb)
        pltpu.sync_copy(dst_h.at[pl.ds(base, ept)], dstb)
        pltpu.sync_copy(ew_h.at[pl.ds(base, ept)], ewb)
        _zero_ref(dego, npad)
        _zero_ref(degi, npad)

        def eb(i, _):
            o = i * L
            sv = srcb[pl.ds(o, L)]
            dv = dstb[pl.ds(o, L)]
            ev = ewb[pl.ds(o, L)]
            plsc.addupdate_scatter(dego, [sv], ev)
            plsc.addupdate_scatter(degi, [dv], ev)
            return 0

        lax.fori_loop(0, ept // L, eb, 0, unroll=4)
        pltpu.sync_copy(dego, dego_h.at[pl.ds(w * npad, npad)])
        pltpu.sync_copy(degi, degi_h.at[pl.ds(w * npad, npad)])

    return deg


def _norms_body(degp_o_ref, degp_i_ref, nsrc_ref, ndst_ref):
    do = jnp.sum(degp_o_ref[...], axis=0)
    di = jnp.sum(degp_i_ref[...], axis=0)
    nsrc_ref[...] = jnp.where(do > 0, lax.rsqrt(do), 0.0)
    ndst_ref[...] = jnp.where(di > 0, lax.rsqrt(di), 0.0)


def _xs_body(x_ref, ns_ref, xs_ref):
    xs_ref[...] = x_ref[...] * ns_ref[...]


def _make_agg1(n, f, e, npad, ept, ch, slc):
    mesh = plsc.VectorSubcoreMesh(core_axis_name="c", subcore_axis_name="s", num_cores=NC, num_subcores=NS)
    nch = ept // ch
    NB = 4
    TAIL = nch % NB
    assert ept % ch == 0 and slc % ch == 0 and TAIL == 2

    @functools.partial(
        pl.kernel,
        mesh=mesh,
        compiler_params=pltpu.CompilerParams(needs_layout_passes=False),
        out_type=jax.ShapeDtypeStruct((NC * npad, f), jnp.float32),
        scratch_types=[
            pltpu.VMEM((ept,), jnp.int32),
            pltpu.VMEM((ept + L,), jnp.float32),
        ] + [pltpu.VMEM((ch,), jnp.int32) for _ in range(NB)]
          + [pltpu.VMEM((ch, f), jnp.float32) for _ in range(NB)]
          + [pltpu.VMEM_SHARED((npad, f), jnp.float32)]
          + [pltpu.SemaphoreType.DMA for _ in range(3 * NB)],
    )
    def agg1(x_h, src_h, dst_h, ew_h, agg_h, srcb, ewb, *bufs):
        dstc = bufs[0:NB]
        rows = bufs[NB:2 * NB]
        sh_agg = bufs[2 * NB]
        gsem = bufs[2 * NB + 1:2 * NB + 1 + NB]
        dsem = bufs[2 * NB + 1 + NB:2 * NB + 1 + 2 * NB]
        ssem = bufs[2 * NB + 1 + 2 * NB:2 * NB + 1 + 3 * NB]
        c = lax.axis_index("c")
        s = lax.axis_index("s")
        w = s * NC + c
        ebase = w * ept
        pltpu.sync_copy(src_h.at[pl.ds(ebase, ept)], srcb)
        pltpu.sync_copy(ew_h.at[pl.ds(ebase, ept)], ewb.at[pl.ds(0, ept)])
        zero16 = jnp.zeros((L,), jnp.float32)

        def zrow(i, _):
            for fb in range(f // L):
                rows[0][i, pl.ds(fb * L, L)] = zero16
            return 0

        lax.fori_loop(0, ch, zrow, 0)
        rbase = s * slc
        for k in range(slc // ch):
            pltpu.sync_copy(rows[0], sh_agg.at[pl.ds(rbase + k * ch, ch)])
        plsc.subcore_barrier()

        def issue(cc, k):
            pltpu.async_copy(
                dst_h.at[pl.ds(ebase + cc * ch, ch)], dstc[k], dsem[k])
            pltpu.async_copy(x_h.at[srcb.at[pl.ds(cc * ch, ch)]],
                             rows[k], gsem[k])

        def wait_in(cc, k):
            pltpu.make_async_copy(
                dst_h.at[pl.ds(ebase + cc * ch, ch)], dstc[k], dsem[k]).wait()
            pltpu.make_async_copy(
                x_h.at[srcb.at[pl.ds(cc * ch, ch)]], rows[k], gsem[k]).wait()

        def drain_sc(k):
            pltpu.make_async_copy(rows[k], sh_agg.at[dstc[k]], ssem[k]).wait()

        def scale(cc, k):
            cbase = cc * ch

            def eb(e2, _):
                cs = ewb[pl.ds(cbase + e2, L)][0]
                for fb in range(f // L):
                    sl = pl.ds(fb * L, L)
                    rows[k][e2, sl] = rows[k][e2, sl] * cs
                return 0

            lax.fori_loop(0, ch, eb, 0, unroll=4)

        for cc in range(NB):
            issue(cc, cc)

        def ring(i, _):
            for k in range(NB):
                cc = NB * i + k
                wait_in(cc, k)
                scale(cc, k)
                pltpu.async_copy(rows[k], sh_agg.at[dstc[k]], ssem[k],
                                 add=True)
                pf = cc + 2
                kj = (k + 2) % NB

                @pl.when(jnp.logical_and(pf >= NB, pf < nch))
                def _():
                    drain_sc(kj)
                    issue(pf, kj)

            return 0

        lax.fori_loop(0, (nch - TAIL) // NB, ring, 0)
        for t in range(TAIL):
            cc = nch - TAIL + t
            wait_in(cc, t)
            scale(cc, t)
            pltpu.sync_copy(rows[t], sh_agg.at[dstc[t]], add=True)
        for k in range(TAIL, NB):
            drain_sc(k)
        plsc.subcore_barrier()
        pltpu.sync_copy(sh_agg.at[pl.ds(rbase, slc)],
                        agg_h.at[pl.ds(c * npad + rbase, slc)])

    return agg1


def _make_agg2(e, npad, ept):
    mesh = plsc.VectorSubcoreMesh(core_axis_name="c", subcore_axis_name="s", num_cores=NC, num_subcores=NS)

    @functools.partial(
        pl.kernel,
        mesh=mesh,
        compiler_params=pltpu.CompilerParams(needs_layout_passes=False),
        out_type=jax.ShapeDtypeStruct((NC * NS * npad,), jnp.float32),
        scratch_types=[
            pltpu.VMEM((npad,), jnp.float32),
            pltpu.VMEM((ept,), jnp.int32),
            pltpu.VMEM((ept,), jnp.int32),
            pltpu.VMEM((ept,), jnp.float32),
            pltpu.VMEM((npad,), jnp.float32),
        ],
    )
    def agg2(z_h, src_h, dst_h, ew_h, out_h,
             zb, srcb, dstb, ewb, accb):
        c = lax.axis_index("c")
        s = lax.axis_index("s")
        w = s * NC + c
        ebase = w * ept
        pltpu.sync_copy(z_h, zb)
        pltpu.sync_copy(src_h.at[pl.ds(ebase, ept)], srcb)
        pltpu.sync_copy(dst_h.at[pl.ds(ebase, ept)], dstb)
        pltpu.sync_copy(ew_h.at[pl.ds(ebase, ept)], ewb)
        _zero_ref(accb, npad)

        def eb(i, _):
            o = i * L
            sv = srcb[pl.ds(o, L)]
            dv = dstb[pl.ds(o, L)]
            ev = ewb[pl.ds(o, L)]
            zg = plsc.load_gather(zb, [sv])
            plsc.addupdate_scatter(accb, [dv], zg * ev)
            return 0

        lax.fori_loop(0, ept // L, eb, 0, unroll=4)
        pltpu.sync_copy(accb, out_h.at[pl.ds(w * npad, npad)])

    return agg2


def _dense1_body(n, aggp_ref, nd_ref, ns_ref, mask_ref, w1_ref, b1_ref,
                 g1_ref, be1_ref, al1_ref, w2_ref, z_ref):
    a = (aggp_ref[0] + aggp_ref[1]) * nd_ref[...]
    h = jnp.dot(a, w1_ref[...], preferred_element_type=jnp.float32)
    h = (h + b1_ref[...]) * mask_ref[...]
    s1 = jnp.sum(h, axis=0, keepdims=True)
    s2 = jnp.sum(h * h, axis=0, keepdims=True)
    al = al1_ref[...]
    mean = s1 * (1.0 / n)
    var = s2 * (1.0 / n) - (2.0 * al) * mean * (s1 * (1.0 / n)) \
        + al * al * mean * mean
    hn = g1_ref[...] * (h - al * mean) * lax.rsqrt(var + EPS) + be1_ref[...]
    hn = jnp.maximum(hn, 0.0)
    z_ref[...] = jnp.dot(
        hn, w2_ref[...], preferred_element_type=jnp.float32) * ns_ref[...]


def _dense2_body(n, accp_ref, nd_ref, mask_ref, b2_ref, g2_ref, be2_ref,
                 al2_ref, out_ref):
    v = jnp.sum(accp_ref[...], axis=0) * nd_ref[...]
    v = (v + b2_ref[0, 0]) * mask_ref[...]
    s1 = jnp.sum(v)
    s2 = jnp.sum(v * v)
    al = al2_ref[0, 0]
    mean = s1 * (1.0 / n)
    var = s2 * (1.0 / n) - (2.0 * al) * mean * (s1 * (1.0 / n)) \
        + al * al * mean * mean
    out_ref[...] = jax.nn.sigmoid(
        g2_ref[0, 0] * (v - al * mean) * lax.rsqrt(var + EPS) + be2_ref[0, 0])


def kernel(inputs, edges, edges_weight, W1, b1, gn1_gamma, gn1_beta, gn1_alpha,
           W2, b2, gn2_gamma, gn2_beta, gn2_alpha):
    x = inputs
    n, f = x.shape
    e = edges.shape[1]
    slc = ((n + NS * L - 1) // (NS * L)) * L
    npad = slc * NS
    assert e % (NC * NS * L) == 0
    src = edges[0].astype(jnp.int32)
    dst = edges[1].astype(jnp.int32)
    ew = edges_weight

    rows2 = npad // 128
    degp_o, degp_i = _make_deg(e, npad, e // (NC * NS))(src, dst, ew)
    nsrc2d, ndst2d = pl.pallas_call(
        _norms_body,
        out_shape=[
            jax.ShapeDtypeStruct((rows2, 128), jnp.float32),
            jax.ShapeDtypeStruct((rows2, 128), jnp.float32),
        ],
    )(degp_o.reshape(NC * NS, rows2, 128), degp_i.reshape(NC * NS, rows2, 128))
    ndst = ndst2d.reshape(npad)
    nscol = nsrc2d.reshape(npad)[:, None]
    xs = pl.pallas_call(
        _xs_body,
        out_shape=jax.ShapeDtypeStruct((n, f), jnp.float32),
    )(x, nscol[:n])

    aggp = _make_agg1(n, f, e, npad, e // (NC * NS), 40, slc)(
        xs, src, dst, ew).reshape(NC, npad, f)

    mask = (jnp.arange(npad) < n).astype(jnp.float32)[:, None]
    z = pl.pallas_call(
        functools.partial(_dense1_body, n),
        out_shape=jax.ShapeDtypeStruct((npad, 1), jnp.float32),
    )(aggp, ndst[:, None], nscol, mask, W1, b1[None, :], gn1_gamma[None, :],
      gn1_beta[None, :], gn1_alpha[None, :], W2)

    acc2p = _make_agg2(e, npad, e // (NC * NS))(
        z.reshape(npad), src, dst, ew)

    out = pl.pallas_call(
        functools.partial(_dense2_body, n),
        out_shape=jax.ShapeDtypeStruct((rows2, 128), jnp.float32),
    )(acc2p.reshape(NC * NS, rows2, 128), ndst.reshape(rows2, 128),
      mask.reshape(rows2, 128), b2[None, :], gn2_gamma[None, :],
      gn2_beta[None, :], gn2_alpha[None, :])
    return out.reshape(npad)[:n, None]


# speedup vs baseline: 1.2139x; 1.2139x over previous
import functools

import jax
import jax.numpy as jnp
from jax import lax
from jax.experimental import pallas as pl
from jax.experimental.pallas import tpu as pltpu
from jax.experimental.pallas import tpu_sc as plsc

EPS = 1e-5
NC = 2
NS = 16
L = 16


def _rsqrt16(x):
    i = plsc.bitcast(x, jnp.int32)
    y = plsc.bitcast(jnp.int32(0x5F3759DF) - (i >> 1), jnp.float32)
    for _ in range(3):
        y = y * (1.5 - 0.5 * x * y * y)
    return jnp.where(x > 0, y, 0.0)


def _zero_ref(ref, nwords):
    zero16 = jnp.zeros((L,), jnp.float32)

    def zb(i, _):
        ref[pl.ds(i * L, L)] = zero16
        return 0

    lax.fori_loop(0, nwords // L, zb, 0)


def _make_deg(e, npad, ept):
    mesh = plsc.VectorSubcoreMesh(core_axis_name="c", subcore_axis_name="s", num_cores=NC, num_subcores=NS)

    @functools.partial(
        pl.kernel,
        mesh=mesh,
        compiler_params=pltpu.CompilerParams(needs_layout_passes=False),
        out_type=[
            jax.ShapeDtypeStruct((NC * NS * npad,), jnp.float32),
            jax.ShapeDtypeStruct((NC * NS * npad,), jnp.float32),
        ],
        scratch_types=[
            pltpu.VMEM((ept,), jnp.int32),
            pltpu.VMEM((ept,), jnp.int32),
            pltpu.VMEM((ept,), jnp.float32),
            pltpu.VMEM((npad,), jnp.float32),
            pltpu.VMEM((npad,), jnp.float32),
        ],
    )
    def deg(src_h, dst_h, ew_h, dego_h, degi_h, srcb, dstb, ewb, dego, degi):
        c = lax.axis_index("c")
        s = lax.axis_index("s")
        w = s * NC + c
        base = w * ept
        pltpu.sync_copy(src_h.at[pl.ds(base, ept)], srcb)
        pltpu.sync_copy(dst_h.at[pl.ds(base, ept)], dstb)
        pltpu.sync_copy(ew_h.at[pl.ds(base, ept)], ewb)
        _zero_ref(dego, npad)
        _zero_ref(degi, npad)

        def eb(i, _):
            o = i * L
            sv = srcb[pl.ds(o, L)]
            dv = dstb[pl.ds(o, L)]
            ev = ewb[pl.ds(o, L)]
            plsc.addupdate_scatter(dego, [sv], ev)
            plsc.addupdate_scatter(degi, [dv], ev)
            return 0

        lax.fori_loop(0, ept // L, eb, 0, unroll=4)
        pltpu.sync_copy(dego, dego_h.at[pl.ds(w * npad, npad)])
        pltpu.sync_copy(degi, degi_h.at[pl.ds(w * npad, npad)])

    return deg


def _norms_body(degp_o_ref, degp_i_ref, nsrc_ref, ndst_ref):
    do = jnp.sum(degp_o_ref[...], axis=0)
    di = jnp.sum(degp_i_ref[...], axis=0)
    nsrc_ref[...] = jnp.where(do > 0, lax.rsqrt(do), 0.0)
    ndst_ref[...] = jnp.where(di > 0, lax.rsqrt(di), 0.0)


def _xs_body(x_ref, ns_ref, xs_ref):
    xs_ref[...] = x_ref[...] * ns_ref[...]


def _make_agg1(n, f, e, npad, ept, ch, slc):
    mesh = plsc.VectorSubcoreMesh(core_axis_name="c", subcore_axis_name="s", num_cores=NC, num_subcores=NS)
    nch = ept // ch
    NB = 3
    TAIL = nch % NB
    assert ept % ch == 0 and slc % ch == 0 and ch % L == 0 and TAIL == 2

    @functools.partial(
        pl.kernel,
        mesh=mesh,
        compiler_params=pltpu.CompilerParams(needs_layout_passes=False),
        out_type=jax.ShapeDtypeStruct((NC * npad, f), jnp.float32),
        scratch_types=[
            pltpu.VMEM((ept,), jnp.int32),
        ] + [pltpu.VMEM((ch,), jnp.int32) for _ in range(NB)]
          + [pltpu.VMEM((ch,), jnp.float32) for _ in range(NB)]
          + [pltpu.VMEM((ch, f), jnp.float32) for _ in range(NB)]
          + [pltpu.VMEM_SHARED((npad, f), jnp.float32)]
          + [pltpu.SemaphoreType.DMA for _ in range(3 * NB)],
    )
    def agg1(x_h, src_h, dst_h, ew_h, agg_h, srcb, *bufs):
        dstc = bufs[0:NB]
        ewc = bufs[NB:2 * NB]
        rows = bufs[2 * NB:3 * NB]
        sh_agg = bufs[3 * NB]
        gsem = bufs[3 * NB + 1:3 * NB + 1 + NB]
        dsem = bufs[3 * NB + 1 + NB:3 * NB + 1 + 2 * NB]
        ssem = bufs[3 * NB + 1 + 2 * NB:3 * NB + 1 + 3 * NB]
        c = lax.axis_index("c")
        s = lax.axis_index("s")
        w = s * NC + c
        ebase = w * ept
        pltpu.sync_copy(src_h.at[pl.ds(ebase, ept)], srcb)
        zero16 = jnp.zeros((L,), jnp.float32)

        def zrow(i, _):
            for fb in range(f // L):
                rows[0][i, pl.ds(fb * L, L)] = zero16
            return 0

        lax.fori_loop(0, ch, zrow, 0)
        rbase = s * slc
        for k in range(slc // ch):
            pltpu.sync_copy(rows[0], sh_agg.at[pl.ds(rbase + k * ch, ch)])
        plsc.subcore_barrier()

        def issue(cc, k):
            pltpu.async_copy(
                dst_h.at[pl.ds(ebase + cc * ch, ch)], dstc[k], dsem[k])
            pltpu.async_copy(
                ew_h.at[pl.ds(ebase + cc * ch, ch)], ewc[k], dsem[k])
            pltpu.async_copy(x_h.at[srcb.at[pl.ds(cc * ch, ch)]],
                             rows[k], gsem[k])

        def wait_in(cc, k):
            pltpu.make_async_copy(
                dst_h.at[pl.ds(ebase + cc * ch, ch)], dstc[k], dsem[k]).wait()
            pltpu.make_async_copy(
                ew_h.at[pl.ds(ebase + cc * ch, ch)], ewc[k], dsem[k]).wait()
            pltpu.make_async_copy(
                x_h.at[srcb.at[pl.ds(cc * ch, ch)]], rows[k], gsem[k]).wait()

        def drain_sc(k):
            pltpu.make_async_copy(rows[k], sh_agg.at[dstc[k]], ssem[k]).wait()

        def scale(k):
            def gb(g, _):
                cv = ewc[k][pl.ds(g * L, L)]
                for j in range(L):
                    spl = cv[j]
                    e2 = g * L + j
                    for fb in range(f // L):
                        sl = pl.ds(fb * L, L)
                        rows[k][e2, sl] = rows[k][e2, sl] * spl
                return 0

            lax.fori_loop(0, ch // L, gb, 0)

        for cc in range(NB):
            issue(cc, cc)

        def ring(i, _):
            for k in range(NB):
                cc = NB * i + k
                wait_in(cc, k)
                scale(k)
                pltpu.async_copy(rows[k], sh_agg.at[dstc[k]], ssem[k],
                                 add=True)
                pf = cc + 2
                kj = (k + 2) % NB

                @pl.when(jnp.logical_and(pf >= NB, pf < nch))
                def _():
                    drain_sc(kj)
                    issue(pf, kj)

            return 0

        lax.fori_loop(0, (nch - TAIL) // NB, ring, 0)
        for t in range(TAIL):
            cc = nch - TAIL + t
            wait_in(cc, t)
            scale(t)
            pltpu.sync_copy(rows[t], sh_agg.at[dstc[t]], add=True)
        for k in range(TAIL, NB):
            drain_sc(k)
        plsc.subcore_barrier()
        pltpu.sync_copy(sh_agg.at[pl.ds(rbase, slc)],
                        agg_h.at[pl.ds(c * npad + rbase, slc)])

    return agg1


def _make_agg2(e, npad, ept):
    mesh = plsc.VectorSubcoreMesh(core_axis_name="c", subcore_axis_name="s", num_cores=NC, num_subcores=NS)

    @functools.partial(
        pl.kernel,
        mesh=mesh,
        compiler_params=pltpu.CompilerParams(needs_layout_passes=False),
        out_type=jax.ShapeDtypeStruct((NC * NS * npad,), jnp.float32),
        scratch_types=[
            pltpu.VMEM((npad,), jnp.float32),
            pltpu.VMEM((ept,), jnp.int32),
            pltpu.VMEM((ept,), jnp.int32),
            pltpu.VMEM((ept,), jnp.float32),
            pltpu.VMEM((npad,), jnp.float32),
        ],
    )
    def agg2(z_h, src_h, dst_h, ew_h, out_h,
             zb, srcb, dstb, ewb, accb):
        c = lax.axis_index("c")
        s = lax.axis_index("s")
        w = s * NC + c
        ebase = w * ept
        pltpu.sync_copy(z_h, zb)
        pltpu.sync_copy(src_h.at[pl.ds(ebase, ept)], srcb)
        pltpu.sync_copy(dst_h.at[pl.ds(ebase, ept)], dstb)
        pltpu.sync_copy(ew_h.at[pl.ds(ebase, ept)], ewb)
        _zero_ref(accb, npad)

        def eb(i, _):
            o = i * L
            sv = srcb[pl.ds(o, L)]
            dv = dstb[pl.ds(o, L)]
            ev = ewb[pl.ds(o, L)]
            zg = plsc.load_gather(zb, [sv])
            plsc.addupdate_scatter(accb, [dv], zg * ev)
            return 0

        lax.fori_loop(0, ept // L, eb, 0, unroll=4)
        pltpu.sync_copy(accb, out_h.at[pl.ds(w * npad, npad)])

    return agg2


def _dense1_body(n, aggp_ref, nd_ref, ns_ref, mask_ref, w1_ref, b1_ref,
                 g1_ref, be1_ref, al1_ref, w2_ref, z_ref):
    a = (aggp_ref[0] + aggp_ref[1]) * nd_ref[...]
    h = jnp.dot(a, w1_ref[...], preferred_element_type=jnp.float32)
    h = (h + b1_ref[...]) * mask_ref[...]
    s1 = jnp.sum(h, axis=0, keepdims=True)
    s2 = jnp.sum(h * h, axis=0, keepdims=True)
    al = al1_ref[...]
    mean = s1 * (1.0 / n)
    var = s2 * (1.0 / n) - (2.0 * al) * mean * (s1 * (1.0 / n)) \
        + al * al * mean * mean
    hn = g1_ref[...] * (h - al * mean) * lax.rsqrt(var + EPS) + be1_ref[...]
    hn = jnp.maximum(hn, 0.0)
    z_ref[...] = jnp.dot(
        hn, w2_ref[...], preferred_element_type=jnp.float32) * ns_ref[...]


def _dense2_body(n, accp_ref, nd_ref, mask_ref, b2_ref, g2_ref, be2_ref,
                 al2_ref, out_ref):
    v = jnp.sum(accp_ref[...], axis=0) * nd_ref[...]
    v = (v + b2_ref[0, 0]) * mask_ref[...]
    s1 = jnp.sum(v)
    s2 = jnp.sum(v * v)
    al = al2_ref[0, 0]
    mean = s1 * (1.0 / n)
    var = s2 * (1.0 / n) - (2.0 * al) * mean * (s1 * (1.0 / n)) \
        + al * al * mean * mean
    out_ref[...] = jax.nn.sigmoid(
        g2_ref[0, 0] * (v - al * mean) * lax.rsqrt(var + EPS) + be2_ref[0, 0])


def kernel(inputs, edges, edges_weight, W1, b1, gn1_gamma, gn1_beta, gn1_alpha,
           W2, b2, gn2_gamma, gn2_beta, gn2_alpha):
    x = inputs
    n, f = x.shape
    e = edges.shape[1]
    slc = ((n + NS * L - 1) // (NS * L)) * L
    npad = slc * NS
    assert e % (NC * NS * L) == 0
    src = edges[0].astype(jnp.int32)
    dst = edges[1].astype(jnp.int32)
    ew = edges_weight

    rows2 = npad // 128
    degp_o, degp_i = _make_deg(e, npad, e // (NC * NS))(src, dst, ew)
    nsrc2d, ndst2d = pl.pallas_call(
        _norms_body,
        out_shape=[
            jax.ShapeDtypeStruct((rows2, 128), jnp.float32),
            jax.ShapeDtypeStruct((rows2, 128), jnp.float32),
        ],
    )(degp_o.reshape(NC * NS, rows2, 128), degp_i.reshape(NC * NS, rows2, 128))
    ndst = ndst2d.reshape(npad)
    nscol = nsrc2d.reshape(npad)[:, None]
    xs = pl.pallas_call(
        _xs_body,
        out_shape=jax.ShapeDtypeStruct((n, f), jnp.float32),
    )(x, nscol[:n])

    aggp = _make_agg1(n, f, e, npad, e // (NC * NS), 80, slc)(
        xs, src, dst, ew).reshape(NC, npad, f)

    mask = (jnp.arange(npad) < n).astype(jnp.float32)[:, None]
    z = pl.pallas_call(
        functools.partial(_dense1_body, n),
        out_shape=jax.ShapeDtypeStruct((npad, 1), jnp.float32),
    )(aggp, ndst[:, None], nscol, mask, W1, b1[None, :], gn1_gamma[None, :],
      gn1_beta[None, :], gn1_alpha[None, :], W2)

    acc2p = _make_agg2(e, npad, e // (NC * NS))(
        z.reshape(npad), src, dst, ew)

    out = pl.pallas_call(
        functools.partial(_dense2_body, n),
        out_shape=jax.ShapeDtypeStruct((rows2, 128), jnp.float32),
    )(acc2p.reshape(NC * NS, rows2, 128), ndst.reshape(rows2, 128),
      mask.reshape(rows2, 128), b2[None, :], gn2_gamma[None, :],
      gn2_beta[None, :], gn2_alpha[None, :])
    return out.reshape(npad)[:n, None]
